# Initial kernel scaffold; baseline (speedup 1.0000x reference)
import jax
import jax.numpy as jnp
from jax.experimental import pallas as pl


def kernel(x, edge_index, edge_attr, batch, atom_emb, bond_emb, eps, W1, b1, W2, b2, Wout, bout):
    raise NotImplementedError("write your pallas kernel here")


# speedup vs baseline: 3.9354x; 3.9354x over previous
import jax
import jax.numpy as jnp
from jax import lax
from jax.experimental import pallas as pl
from jax.experimental.pallas import tpu as pltpu
from jax.experimental.pallas import tpu_sc as plsc

F32 = jnp.float32
I32 = jnp.int32

HID = 128
LAYERS = 4
NGRAPH = 64
NNODES = 10000
NEDGES = 320000
NPAD = 10240
NCHUNK = NPAD // 128
NW = 32
ECH = 64
ECH_PER_W = 162
EPAD = NW * ECH_PER_W * ECH
TRASH = 10016
ROWS_PER_TILE = NPAD // 16

_MESH = dict(core_axis_name="c", subcore_axis_name="s", num_cores=2,
             num_subcores=16)


def _sc_msg_body(h_hbm, et_hbm, src_hbm, dst_hbm, combo_hbm, agg_hbm,
                 agg_sh, hb, eb, si, di, ci, sd,
                 sem_i0, sem_i1, sem_h0, sem_h1, sem_h2,
                 sem_e0, sem_e1, sem_s0, sem_s1, sem_s2):
    cid = lax.axis_index("c")
    sid = lax.axis_index("s")
    w = sid * 2 + cid
    sems_i = (sem_i0, sem_i1)
    sems_h = (sem_h0, sem_h1, sem_h2)
    sems_e = (sem_e0, sem_e1)
    sems_s = (sem_s0, sem_s1, sem_s2)

    z0 = hb.at[0]

    @pl.loop(0, ECH)
    def _zero_fill(i):
        for j in range(8):
            z0[i, pl.ds(j * 16, 16)] = jnp.zeros((16,), F32)

    @pl.loop(0, ROWS_PER_TILE // ECH)
    def _zero_out(k):
        pltpu.sync_copy(z0, agg_sh.at[pl.ds(sid * ROWS_PER_TILE + k * ECH,
                                            ECH)])

    plsc.subcore_barrier()

    def start_idx(c, b):
        pltpu.async_copy(src_hbm.at[w, c], si.at[b], sems_i[b])
        pltpu.async_copy(dst_hbm.at[w, c], di.at[b], sems_i[b])
        pltpu.async_copy(combo_hbm.at[w, c], ci.at[b], sems_i[b])

    def wait_idx(c, b):
        pltpu.make_async_copy(src_hbm.at[w, c], si.at[b], sems_i[b]).wait()
        pltpu.make_async_copy(dst_hbm.at[w, c], di.at[b], sems_i[b]).wait()
        pltpu.make_async_copy(combo_hbm.at[w, c], ci.at[b], sems_i[b]).wait()

    def start_gather(b2, b3):
        pltpu.async_copy(h_hbm.at[si.at[b2]], hb.at[b3], sems_h[b3])
        pltpu.async_copy(et_hbm.at[ci.at[b2]], eb.at[b2], sems_e[b2])

    def wait_gather(b2, b3):
        pltpu.make_async_copy(h_hbm.at[si.at[b2]], hb.at[b3],
                              sems_h[b3]).wait()
        pltpu.make_async_copy(et_hbm.at[ci.at[b2]], eb.at[b2],
                              sems_e[b2]).wait()

    def wait_scatter(b3):
        pltpu.make_async_copy(hb.at[b3], agg_sh.at[sd.at[b3]],
                              sems_s[b3]).wait()

    pltpu.sync_copy(src_hbm.at[w, 0], si.at[0])
    pltpu.sync_copy(dst_hbm.at[w, 0], di.at[0])
    pltpu.sync_copy(combo_hbm.at[w, 0], ci.at[0])
    start_idx(1, 1)
    start_gather(0, 0)

    @pl.loop(0, ECH_PER_W // 6)
    def _six(q):
        for b6 in range(6):
            c = q * 6 + b6
            i2, i3 = b6 % 2, b6 % 3
            n2, n3 = (b6 + 1) % 2, (b6 + 1) % 3

            @pl.when(c < ECH_PER_W - 1)
            def _():
                wait_idx(c + 1, n2)

                @pl.when(c >= 2)
                def _():
                    wait_scatter(n3)

                start_gather(n2, n3)

            wait_gather(i2, i3)
            hbk = hb.at[i3]
            ebk = eb.at[i2]

            @pl.loop(0, ECH, unroll=4)
            def _row(i):
                for j in range(8):
                    s = pl.ds(j * 16, 16)
                    hbk[i, s] = jnp.maximum(hbk[i, s] + ebk[i, s], 0.0)

            for t in range(ECH // 16):
                sd[i3, pl.ds(t * 16, 16)] = di[i2, pl.ds(t * 16, 16)]
            pltpu.async_copy(hbk, agg_sh.at[sd.at[i3]], sems_s[i3], add=True)

            @pl.when(c < ECH_PER_W - 2)
            def _():
                start_idx(c + 2, i2)

    for k in range(3):
        wait_scatter(k)
    plsc.subcore_barrier()

    nfl = ROWS_PER_TILE // ECH
    for k in range(nfl):
        b = k % 2
        if k >= 2:
            pltpu.make_async_copy(
                hb.at[b], agg_hbm.at[cid, pl.ds(0, ECH)], sems_h[b]).wait()
        r = sid * ROWS_PER_TILE + k * ECH
        pltpu.sync_copy(agg_sh.at[pl.ds(r, ECH)], hb.at[b])
        pltpu.async_copy(hb.at[b], agg_hbm.at[cid, pl.ds(r, ECH)], sems_h[b])
    for k in range(nfl - 2, nfl):
        b = k % 2
        pltpu.make_async_copy(
            hb.at[b], agg_hbm.at[cid, pl.ds(0, ECH)], sems_h[b]).wait()


def _sc_msg(h, et, src, dst, combo):
    return pl.kernel(
        _sc_msg_body,
        out_type=jax.ShapeDtypeStruct((2, NPAD, HID), F32),
        mesh=plsc.VectorSubcoreMesh(**_MESH),
        scratch_types=[
            pltpu.VMEM_SHARED((NPAD, HID), F32),
            pltpu.VMEM((3, ECH, HID), F32),
            pltpu.VMEM((2, ECH, HID), F32),
            pltpu.VMEM((2, ECH), I32),
            pltpu.VMEM((2, ECH), I32),
            pltpu.VMEM((2, ECH), I32),
            pltpu.VMEM((3, ECH), I32),
            pltpu.SemaphoreType.DMA,
            pltpu.SemaphoreType.DMA,
            pltpu.SemaphoreType.DMA,
            pltpu.SemaphoreType.DMA,
            pltpu.SemaphoreType.DMA,
            pltpu.SemaphoreType.DMA,
            pltpu.SemaphoreType.DMA,
            pltpu.SemaphoreType.DMA,
            pltpu.SemaphoreType.DMA,
            pltpu.SemaphoreType.DMA,
        ],
    )(h, et, src, dst, combo)


def _sc_pool_body(h_hbm, bq_hbm, sums_hbm, counts_hbm,
                  sums_sh, counts_sh, hbuf, ones_v, bidx, sem):
    del sem
    cid = lax.axis_index("c")
    sid = lax.axis_index("s")
    w = sid * 2 + cid

    @pl.loop(0, 128)
    def _fill(i):
        for j in range(8):
            s = pl.ds(j * 16, 16)
            hbuf[i, s] = jnp.zeros((16,), F32)
            ones_v[i, s] = jnp.full((16,), 1.0, F32)

    pltpu.sync_copy(hbuf.at[pl.ds(0, 5)], sums_sh.at[pl.ds(sid * 5, 5)])
    pltpu.sync_copy(hbuf.at[pl.ds(0, 5)], counts_sh.at[pl.ds(sid * 5, 5)])
    plsc.subcore_barrier()

    for t in range(3):
        cc = w + NW * t

        @pl.when(cc < NCHUNK)
        def _():
            pltpu.sync_copy(h_hbm.at[pl.ds(cc * 128, 128)], hbuf)
            pltpu.sync_copy(bq_hbm.at[cc], bidx)
            pltpu.sync_copy(hbuf, sums_sh.at[bidx], add=True)
            pltpu.sync_copy(ones_v, counts_sh.at[bidx], add=True)

    plsc.subcore_barrier()

    @pl.when(sid == 0)
    def _():
        pltpu.sync_copy(sums_sh, hbuf.at[pl.ds(0, 80)])
        pltpu.sync_copy(hbuf.at[pl.ds(0, 80)], sums_hbm.at[cid])
        pltpu.sync_copy(counts_sh, ones_v.at[pl.ds(0, 80)])
        pltpu.sync_copy(ones_v.at[pl.ds(0, 80)], counts_hbm.at[cid])


def _sc_pool(h, bq):
    return pl.kernel(
        _sc_pool_body,
        out_type=(jax.ShapeDtypeStruct((2, NCHUNK, HID), F32),
                  jax.ShapeDtypeStruct((2, NCHUNK, HID), F32)),
        mesh=plsc.VectorSubcoreMesh(**_MESH),
        scratch_types=[
            pltpu.VMEM_SHARED((NCHUNK, HID), F32),
            pltpu.VMEM_SHARED((NCHUNK, HID), F32),
            pltpu.VMEM((128, HID), F32),
            pltpu.VMEM((128, HID), F32),
            pltpu.VMEM((128,), I32),
            pltpu.SemaphoreType.DMA,
        ],
    )(h, bq)


def _tc_atom_body(xt_ref, at_ref, out_ref):
    col = lax.broadcasted_iota(I32, (1280, 128), 1)
    acc = jnp.zeros((1280, 128), F32)
    for f in range(9):
        oh = (xt_ref[f, :][:, None] == col).astype(F32)
        acc = acc + jnp.dot(oh, at_ref[f], preferred_element_type=F32)
    out_ref[...] = acc


def _tc_atom(xt, atom_pad):
    return pl.pallas_call(
        _tc_atom_body,
        grid=(8,),
        in_specs=[pl.BlockSpec((16, 1280), lambda i: (0, i)),
                  pl.BlockSpec((9, 128, HID), lambda i: (0, 0, 0))],
        out_specs=pl.BlockSpec((1280, HID), lambda i: (i, 0)),
        out_shape=jax.ShapeDtypeStruct((NPAD, HID), F32),
    )(xt, atom_pad)


def _tc_bond_body(bf_ref, out_ref):
    fl = lax.broadcasted_iota(I32, (512, 64), 0)
    col = lax.broadcasted_iota(I32, (512, 64), 1)
    l = fl // 128
    c = jnp.minimum(fl % 128, 124)
    c0 = c // 25
    c1 = (c // 5) % 5
    c2 = c % 5
    m = ((col == l * 15 + c0) | (col == l * 15 + 5 + c1)
         | (col == l * 15 + 10 + c2))
    out_ref[...] = jnp.dot(m.astype(F32), bf_ref[...],
                           preferred_element_type=F32)


def _tc_bond(bond_pad):
    return pl.pallas_call(
        _tc_bond_body,
        grid=(1,),
        in_specs=[pl.BlockSpec((64, HID), lambda i: (0, 0))],
        out_specs=pl.BlockSpec((512, HID), lambda i: (0, 0)),
        out_shape=jax.ShapeDtypeStruct((4 * 128, HID), F32),
    )(bond_pad)


def _tc_combo_body(ea_ref, out_ref):
    combo = ea_ref[0, :] * 25 + ea_ref[1, :] * 5 + ea_ref[2, :]
    out_ref[...] = combo.reshape(32, 128)


def _tc_combo(ea_t):
    return pl.pallas_call(
        _tc_combo_body,
        grid=(EPAD // 4096,),
        in_specs=[pl.BlockSpec((8, 4096), lambda i: (0, i))],
        out_specs=pl.BlockSpec((32, 128), lambda i: (i, 0)),
        out_shape=jax.ShapeDtypeStruct((EPAD // 128, 128), I32),
    )(ea_t)


def _tc_mlp_body(h_ref, agg_ref, epsb_ref, w1_ref, b1_ref, w2_ref, b2_ref,
                 out_ref):
    a = agg_ref[0] + agg_ref[1]
    z0 = h_ref[...] * epsb_ref[...] + a
    z = jnp.maximum(jnp.dot(z0, w1_ref[...], preferred_element_type=F32)
                    + b1_ref[...], 0.0)
    out_ref[...] = jnp.maximum(
        jnp.dot(z, w2_ref[...], preferred_element_type=F32) + b2_ref[...],
        0.0)


def _tc_mlp(h, agg, epsb, w1, b1, w2, b2):
    return pl.pallas_call(
        _tc_mlp_body,
        grid=(8,),
        in_specs=[
            pl.BlockSpec((1280, HID), lambda i: (i, 0)),
            pl.BlockSpec((2, 1280, HID), lambda i: (0, i, 0)),
            pl.BlockSpec((1, HID), lambda i: (0, 0)),
            pl.BlockSpec((HID, 2 * HID), lambda i: (0, 0)),
            pl.BlockSpec((1, 2 * HID), lambda i: (0, 0)),
            pl.BlockSpec((2 * HID, HID), lambda i: (0, 0)),
            pl.BlockSpec((1, HID), lambda i: (0, 0)),
        ],
        out_specs=pl.BlockSpec((1280, HID), lambda i: (i, 0)),
        out_shape=jax.ShapeDtypeStruct((NPAD, HID), F32),
    )(h, agg, epsb, w1, b1, w2, b2)


def _tc_head_body(sums_ref, counts_ref, wout_ref, bout_ref, out_ref):
    s = sums_ref[0] + sums_ref[1]
    cnt = counts_ref[0] + counts_ref[1]
    hg = s / jnp.maximum(cnt, 1.0)
    out_ref[...] = (jnp.dot(hg[0:64, :], wout_ref[...],
                            preferred_element_type=F32) + bout_ref[...])


def _tc_head(sums, counts, wout, bout2):
    return pl.pallas_call(
        _tc_head_body,
        grid=(1,),
        in_specs=[
            pl.BlockSpec((2, NCHUNK, HID), lambda i: (0, 0, 0)),
            pl.BlockSpec((2, NCHUNK, HID), lambda i: (0, 0, 0)),
            pl.BlockSpec((HID, HID), lambda i: (0, 0)),
            pl.BlockSpec((1, HID), lambda i: (0, 0)),
        ],
        out_specs=pl.BlockSpec((NGRAPH, HID), lambda i: (0, 0)),
        out_shape=jax.ShapeDtypeStruct((NGRAPH, HID), F32),
    )(sums, counts, wout, bout2)


def kernel(x, edge_index, edge_attr, batch, atom_emb, bond_emb, eps,
           W1, b1, W2, b2, Wout, bout):
    xt = jnp.pad(x.astype(I32).T, ((0, 7), (0, NPAD - NNODES)),
                 constant_values=200)
    atom_pad = jnp.pad(atom_emb.astype(F32), ((0, 0), (0, 9), (0, 0)))
    bond_pad = jnp.pad(bond_emb.astype(F32).reshape(60, HID),
                       ((0, 4), (0, 0)))
    src = jnp.pad(edge_index[0].astype(I32),
                  (0, EPAD - NEDGES)).reshape(NW, ECH_PER_W, ECH)
    dst = jnp.pad(edge_index[1].astype(I32), (0, EPAD - NEDGES),
                  constant_values=TRASH).reshape(NW, ECH_PER_W, ECH)
    ea_t = jnp.pad(edge_attr.astype(I32).T,
                   ((0, 5), (0, EPAD - NEDGES)))
    bq = jnp.pad(batch.astype(I32), (0, NPAD - NNODES),
                 constant_values=NGRAPH).reshape(NCHUNK, 128)
    epsb = jnp.broadcast_to((1.0 + eps.astype(F32))[:, None],
                            (LAYERS, HID)).reshape(LAYERS, 1, HID)
    bout2 = bout.astype(F32).reshape(1, HID)

    h = _tc_atom(xt, atom_pad)
    e_tab = _tc_bond(bond_pad)
    combo = _tc_combo(ea_t).reshape(NW, ECH_PER_W, ECH)

    for l in range(LAYERS):
        et_l = lax.slice_in_dim(e_tab, l * 128, (l + 1) * 128, axis=0)
        agg = _sc_msg(h, et_l, src, dst, combo)
        h = _tc_mlp(h, agg, epsb[l], W1[l].astype(F32),
                    b1[l].reshape(1, 2 * HID).astype(F32),
                    W2[l].astype(F32), b2[l].reshape(1, HID).astype(F32))

    sums, counts = _sc_pool(h, bq)
    return _tc_head(sums, counts, Wout.astype(F32), bout2)
